# baseline (speedup 1.0000x reference)
import functools

import jax
import jax.numpy as jnp
from jax import lax
from jax.experimental import pallas as pl
from jax.experimental.pallas import tpu as pltpu
from jax.experimental.pallas import tpu_sc as plsc

VOCAB = 100000
BATCH = 32
Q_LEN = 8
ROWS = BATCH * Q_LEN
BB = 2


def _fill_body(ls_ref, ls_sq_ref, ids_sq_ref, out_ref, peaks_ref, pos_ref):
    i = pl.program_id(0)
    alpha = jax.nn.sigmoid(ls_ref[pl.ds(i * BB, BB), :])
    base = (1.0 - alpha) * jnp.float32(1.0 / VOCAB)
    log_base = jnp.maximum(jnp.log(base), jnp.float32(-1e6))
    out_ref[...] = jnp.broadcast_to(log_base[..., None], (BB, Q_LEN, VOCAB))

    @pl.when(i == 0)
    def _():
        a_sq = jax.nn.sigmoid(ls_sq_ref[...])
        b_sq = (1.0 - a_sq) * jnp.float32(1.0 / VOCAB)
        peaks_ref[...] = jnp.maximum(jnp.log(b_sq + a_sq), jnp.float32(-1e6))
        r = (
            lax.broadcasted_iota(jnp.int32, (16, 16), 0) * 16
            + lax.broadcasted_iota(jnp.int32, (16, 16), 1)
        )
        pos_ref[...] = r * VOCAB + ids_sq_ref[...]


def _sc_scatter(peaks_hbm, pos_hbm, out_ref, peaks_v, pos_v, sem):
    info = plsc.get_sparse_core_info()
    nc = info.num_cores
    wid = lax.axis_index("s") * nc + lax.axis_index("c")
    pltpu.sync_copy(peaks_hbm.at[wid // 2, pl.ds((wid % 2) * 8, 8)], peaks_v)
    pltpu.sync_copy(pos_hbm.at[wid // 2, pl.ds((wid % 2) * 8, 8)], pos_v)
    pltpu.async_copy(peaks_v, out_ref.at[pos_v], sem).wait()


@jax.jit
def kernel(log_snr, input_ids):
    filled, peaks_sq, pos_sq = pl.pallas_call(
        _fill_body,
        grid=(BATCH // BB,),
        in_specs=[
            pl.BlockSpec((BATCH, Q_LEN), lambda i: (0, 0)),
            pl.BlockSpec((16, 16), lambda i: (0, 0)),
            pl.BlockSpec((16, 16), lambda i: (0, 0)),
        ],
        out_specs=[
            pl.BlockSpec((BB, Q_LEN, VOCAB), lambda i: (i, 0, 0)),
            pl.BlockSpec((16, 16), lambda i: (0, 0)),
            pl.BlockSpec((16, 16), lambda i: (0, 0)),
        ],
        out_shape=[
            jax.ShapeDtypeStruct((BATCH, Q_LEN, VOCAB), jnp.float32),
            jax.ShapeDtypeStruct((16, 16), jnp.float32),
            jax.ShapeDtypeStruct((16, 16), jnp.int32),
        ],
    )(
        log_snr,
        log_snr.reshape(16, 16),
        input_ids.astype(jnp.int32).reshape(16, 16),
    )

    out_ref = jax.new_ref(filled.reshape(ROWS * VOCAB))
    mesh = plsc.VectorSubcoreMesh(core_axis_name="c", subcore_axis_name="s")
    scatter = functools.partial(
        pl.kernel,
        mesh=mesh,
        scratch_types=[
            pltpu.VMEM((8,), jnp.float32),
            pltpu.VMEM((8,), jnp.int32),
            pltpu.SemaphoreType.DMA,
        ],
    )(_sc_scatter)
    scatter(peaks_sq, pos_sq, out_ref)
    return jax.freeze(out_ref).reshape(BATCH, Q_LEN, VOCAB)


# speedup vs baseline: 9.3514x; 9.3514x over previous
import functools

import jax
import jax.numpy as jnp
from jax import lax
from jax.experimental import pallas as pl
from jax.experimental.pallas import tpu as pltpu
from jax.experimental.pallas import tpu_sc as plsc

VOCAB = 100000
BATCH = 32
Q_LEN = 8
ROWS = BATCH * Q_LEN
BB = 2


def _fill_body(ls_ref, ids_ref, out_ref):
    i = pl.program_id(0)
    alpha = jax.nn.sigmoid(ls_ref[pl.ds(i * BB, BB), :])
    base = (1.0 - alpha) * jnp.float32(1.0 / VOCAB)
    log_base = jnp.maximum(jnp.log(base), jnp.float32(-1e6))
    log_peak = jnp.maximum(jnp.log(base + alpha), jnp.float32(-1e6))
    col = jax.lax.broadcasted_iota(jnp.int32, (BB, Q_LEN, VOCAB), 2)
    mask = col == ids_ref[pl.ds(i * BB, BB), :][..., None]
    out_ref[...] = jnp.where(mask, log_peak[..., None], log_base[..., None])


@jax.jit
def kernel(log_snr, input_ids):
    filled = pl.pallas_call(
        _fill_body,
        grid=(BATCH // BB,),
        in_specs=[
            pl.BlockSpec((BATCH, Q_LEN), lambda i: (0, 0)),
            pl.BlockSpec((BATCH, Q_LEN), lambda i: (0, 0)),
        ],
        out_specs=pl.BlockSpec((BB, Q_LEN, VOCAB), lambda i: (i, 0, 0)),
        out_shape=jax.ShapeDtypeStruct((BATCH, Q_LEN, VOCAB), jnp.float32),
    )(log_snr, input_ids.astype(jnp.int32))
    out_ref = jax.new_ref(filled)
    return jax.freeze(out_ref)
